# Initial kernel scaffold; baseline (speedup 1.0000x reference)
import jax
import jax.numpy as jnp
from jax.experimental import pallas as pl


def kernel(x1, cent1, rw1, src1, dst1, batch1, x2, cent2, rw2, src2, dst2, batch2, degree_emb, W_init, b_init, Wc1, bc1, Wc2, bc2, Wc3, bc3, A_aff, Ws1, bs1, Ws2, bs2):
    raise NotImplementedError("write your pallas kernel here")


# speedup vs baseline: 10.2590x; 10.2590x over previous
import math

import jax
import jax.numpy as jnp
from jax.experimental import pallas as pl
from jax.experimental.pallas import tpu as pltpu

_B = 512
_NPG = 64
_EPG = 1024
_XDIM = 29
_MAXDEG = 83
_RWDIM = 16
_F = 128

_LOG_MU = -math.log(4096.0)
_LOG_NU0 = math.log(4088.0 / 4096.0)
_LOG_NU1 = math.log(8.0 / 4096.0)


def _pair_body(x1_r, c1_r, rw1_r, s1_r, d1_r,
               x2_r, c2_r, rw2_r, s2_r, d2_r,
               emb_r, Wx_r, We_r, Wr_r, b0_r,
               Wc1_r, bc1_r, Wc2_r, bc2_r, Wc3_r, bc3_r,
               Aaff_r, Ws1_r, bs1_r, Ws2_r, bs2_r,
               ged_r, sim1_r, sim2_r):
    f32 = jnp.float32

    def embed(x_r, c_r, rw_r):
        cent = c_r[...]
        oh = (cent == jax.lax.broadcasted_iota(jnp.int32, (_NPG, _MAXDEG), 1)
              ).astype(f32)
        e = (oh @ emb_r[...]) @ We_r[...]
        f = x_r[...] @ Wx_r[...] + e + rw_r[...] @ Wr_r[...] + b0_r[...]
        return jnp.maximum(f, 0.0)

    def adjacency(s_r, d_r):
        s = s_r[0] & 63
        d = d_r[0] & 63
        rows = jax.lax.broadcasted_iota(jnp.int32, (_NPG, _EPG), 0)
        ohS = (jnp.broadcast_to(s, (_NPG, _EPG)) == rows).astype(f32)
        ohD = (jnp.broadcast_to(d, (_NPG, _EPG)) == rows).astype(f32)
        counts = jax.lax.dot_general(ohD, ohS, (((1,), (1,)), ((), ())),
                                     preferred_element_type=f32)
        deg = jnp.sum(ohD, axis=1, keepdims=True) + 1.0
        dinv = jax.lax.rsqrt(deg)
        dd = 1.0 / deg
        return counts, dinv, dd

    def gcn(h, counts, dinv, dd, W_r, b_r):
        hw = h @ W_r[...]
        agg = jax.lax.dot_general(counts, hw * dinv, (((1,), (0,)), ((), ())),
                                  preferred_element_type=f32)
        return agg * dinv + hw * dd + b_r[...]

    feat1 = embed(x1_r, c1_r, rw1_r)
    feat2 = embed(x2_r, c2_r, rw2_r)
    counts1, dinv1, dd1 = adjacency(s1_r, d1_r)
    counts2, dinv2, dd2 = adjacency(s2_r, d2_r)

    def conv(feat, counts, dinv, dd):
        a1 = gcn(feat, counts, dinv, dd, Wc1_r, bc1_r)
        a2 = gcn(jnp.maximum(a1, 0.0), counts, dinv, dd, Wc2_r, bc2_r)
        a3 = gcn(jnp.maximum(a2, 0.0), counts, dinv, dd, Wc3_r, bc3_r)
        return a1, a2, a3

    a11, a12, a13 = conv(feat1, counts1, dinv1, dd1)
    a21, a22, a23 = conv(feat2, counts2, dinv2, dd2)

    def sim(xd, yd):
        t = xd @ Aaff_r[...]
        return jax.lax.dot_general(t, yd, (((1,), (1,)), ((), ())),
                                   preferred_element_type=f32)

    def soft_topk(M):
        smin = jnp.min(M)
        smax = jnp.max(M)
        s = (M - smin) / (smax - smin + 1e-8)
        k0 = -(s * s)
        k1 = -((s - 1.0) * (s - 1.0))
        v0 = 0.0
        v1 = 0.0
        u = None
        for _ in range(6):
            a = k0 + v0
            b = k1 + v1
            mx = jnp.maximum(a, b)
            u = _LOG_MU - (mx + jnp.log(jnp.exp(a - mx) + jnp.exp(b - mx)))
            t0 = k0 + u
            t1 = k1 + u
            m0 = jnp.max(t0)
            m1 = jnp.max(t1)
            v0 = _LOG_NU0 - (m0 + jnp.log(jnp.sum(jnp.exp(t0 - m0))))
            v1 = _LOG_NU1 - (m1 + jnp.log(jnp.sum(jnp.exp(t1 - m1))))
        return jnp.exp(k1 + u + v1) * 4096.0

    sim1_r[...] = soft_topk(sim(feat1, feat2))[None]
    sim2_r[...] = soft_topk(sim(a13, a23))[None]

    pooled = [jnp.sum(t, axis=0, keepdims=True)
              for t in (feat1, a11, a12, a13)]
    pooled += [jnp.max(t, axis=0, keepdims=True)
               for t in (feat2, a21, a22, a23)]
    h = bs1_r[...]
    for j, t in enumerate(pooled):
        h = h + t @ Ws1_r[j * _F:(j + 1) * _F, :]
    h = jnp.maximum(h, 0.0)
    g = h @ Ws2_r[...] + bs2_r[...]
    ged_r[...] = jnp.broadcast_to(jax.nn.sigmoid(g), (1, 1, 128))


def kernel(x1, cent1, rw1, src1, dst1, batch1,
           x2, cent2, rw2, src2, dst2, batch2,
           degree_emb, W_init, b_init, Wc1, bc1, Wc2, bc2, Wc3, bc3,
           A_aff, Ws1, bs1, Ws2, bs2):
    del batch1, batch2
    f32 = jnp.float32
    Wx = W_init[:_XDIM]
    We = W_init[_XDIM:_XDIM + _MAXDEG]
    Wr = W_init[_XDIM + _MAXDEG:]
    b0 = b_init.reshape(1, _F)
    bc1r = bc1.reshape(1, _F)
    bc2r = bc2.reshape(1, _F)
    bc3r = bc3.reshape(1, _F)
    bs1r = bs1.reshape(1, 64)
    bs2r = bs2.reshape(1, 1)
    s1 = src1.reshape(_B, 1, _EPG)
    d1 = dst1.reshape(_B, 1, _EPG)
    s2 = src2.reshape(_B, 1, _EPG)
    d2 = dst2.reshape(_B, 1, _EPG)

    node = lambda dim: pl.BlockSpec((_NPG, dim), lambda i: (i, 0))
    edge = pl.BlockSpec((1, 1, _EPG), lambda i: (i, 0, 0))
    full = lambda a: pl.BlockSpec(a.shape, lambda i: (0,) * a.ndim)

    in_specs = [
        node(_XDIM), node(1), node(_RWDIM), edge, edge,
        node(_XDIM), node(1), node(_RWDIM), edge, edge,
        full(degree_emb), full(Wx), full(We), full(Wr), full(b0),
        full(Wc1), full(bc1r), full(Wc2), full(bc2r), full(Wc3), full(bc3r),
        full(A_aff), full(Ws1), full(bs1r), full(Ws2), full(bs2r),
    ]
    out_specs = [
        pl.BlockSpec((1, 1, 128), lambda i: (i, 0, 0)),
        pl.BlockSpec((1, _NPG, _NPG), lambda i: (i, 0, 0)),
        pl.BlockSpec((1, _NPG, _NPG), lambda i: (i, 0, 0)),
    ]
    out_shape = [
        jax.ShapeDtypeStruct((_B, 1, 128), f32),
        jax.ShapeDtypeStruct((_B, _NPG, _NPG), f32),
        jax.ShapeDtypeStruct((_B, _NPG, _NPG), f32),
    ]

    ged3, sim1, sim2 = pl.pallas_call(
        _pair_body,
        grid=(_B,),
        in_specs=in_specs,
        out_specs=out_specs,
        out_shape=out_shape,
        compiler_params=pltpu.CompilerParams(
            dimension_semantics=("arbitrary",)),
    )(x1, cent1, rw1, s1, d1, x2, cent2, rw2, s2, d2,
      degree_emb, Wx, We, Wr, b0, Wc1, bc1r, Wc2, bc2r, Wc3, bc3r,
      A_aff, Ws1, bs1r, Ws2, bs2r)
    return ged3[:, 0, 0], sim1, sim2


# speedup vs baseline: 20.6236x; 2.0103x over previous
import math

import jax
import jax.numpy as jnp
from jax.experimental import pallas as pl
from jax.experimental.pallas import tpu as pltpu

_B = 512
_G = 4
_NPG = 64
_EPG = 1024
_XDIM = 29
_MAXDEG = 83
_RWDIM = 16
_F = 128

_MU = 1.0 / 4096.0
_NU0 = 4088.0 / 4096.0
_NU1 = 8.0 / 4096.0


def _group_body(x1_r, c1_r, rw1_r, s1_r, d1_r,
                x2_r, c2_r, rw2_r, s2_r, d2_r,
                emb_r, Wx_r, We_r, Wr_r, b0_r,
                Wc1_r, bc1_r, Wc2_r, bc2_r, Wc3_r, bc3_r,
                Aaff_r, Ws1_r, bs1_r, Ws2_r, bs2_r,
                ged_r, sim1_r, sim2_r):
    f32 = jnp.float32

    def embed(x, cent, rw):
        oh = (cent == jax.lax.broadcasted_iota(jnp.int32, (_NPG, _MAXDEG), 1)
              ).astype(f32)
        e = (oh @ emb_r[...]) @ We_r[...]
        f = x @ Wx_r[...] + e + rw @ Wr_r[...] + b0_r[...]
        return jnp.maximum(f, 0.0)

    def adjacency(s, d):
        rows = jax.lax.broadcasted_iota(jnp.int32, (_NPG, _EPG), 0)
        ohS = (jnp.broadcast_to(s & 63, (_NPG, _EPG)) == rows).astype(f32)
        ohD = (jnp.broadcast_to(d & 63, (_NPG, _EPG)) == rows).astype(f32)
        counts = jax.lax.dot_general(ohD, ohS, (((1,), (1,)), ((), ())),
                                     preferred_element_type=f32,
                                     precision=jax.lax.Precision.DEFAULT)
        deg = jnp.sum(ohD, axis=1, keepdims=True) + 1.0
        dinv = jax.lax.rsqrt(deg)
        dd = 1.0 / deg
        return counts, dinv, dd

    def gcn(h, counts, dinv, dd, W_r, b_r):
        hw = h @ W_r[...]
        agg = jax.lax.dot_general(counts, hw * dinv, (((1,), (0,)), ((), ())),
                                  preferred_element_type=f32)
        return agg * dinv + hw * dd + b_r[...]

    def sim(xd, yd):
        t = xd @ Aaff_r[...]
        return jax.lax.dot_general(t, yd, (((1,), (1,)), ((), ())),
                                   preferred_element_type=f32)

    def soft_topk(M):
        smin = jnp.min(M)
        smax = jnp.max(M)
        s = (M - smin) / (smax - smin + 1e-8)
        E0 = jnp.exp(-(s * s))
        E1 = jnp.exp(-((s - 1.0) * (s - 1.0)))
        w0 = 1.0
        w1 = 1.0
        U = None
        for _ in range(6):
            U = _MU / (E0 * w0 + E1 * w1)
            w0 = _NU0 / jnp.sum(E0 * U)
            w1 = _NU1 / jnp.sum(E1 * U)
        return E1 * U * (w1 * 4096.0)

    for g in range(_G):
        r0 = g * _NPG
        feat1 = embed(x1_r[r0:r0 + _NPG], c1_r[r0:r0 + _NPG],
                      rw1_r[r0:r0 + _NPG])
        feat2 = embed(x2_r[r0:r0 + _NPG], c2_r[r0:r0 + _NPG],
                      rw2_r[r0:r0 + _NPG])
        counts1, dinv1, dd1 = adjacency(s1_r[g], d1_r[g])
        counts2, dinv2, dd2 = adjacency(s2_r[g], d2_r[g])

        a11 = gcn(feat1, counts1, dinv1, dd1, Wc1_r, bc1_r)
        a12 = gcn(jnp.maximum(a11, 0.0), counts1, dinv1, dd1, Wc2_r, bc2_r)
        a13 = gcn(jnp.maximum(a12, 0.0), counts1, dinv1, dd1, Wc3_r, bc3_r)
        a21 = gcn(feat2, counts2, dinv2, dd2, Wc1_r, bc1_r)
        a22 = gcn(jnp.maximum(a21, 0.0), counts2, dinv2, dd2, Wc2_r, bc2_r)
        a23 = gcn(jnp.maximum(a22, 0.0), counts2, dinv2, dd2, Wc3_r, bc3_r)

        sim1_r[g] = soft_topk(sim(feat1, feat2))
        sim2_r[g] = soft_topk(sim(a13, a23))

        pooled = [jnp.sum(t, axis=0, keepdims=True)
                  for t in (feat1, a11, a12, a13)]
        pooled += [jnp.max(t, axis=0, keepdims=True)
                   for t in (feat2, a21, a22, a23)]
        h = bs1_r[...]
        for j, t in enumerate(pooled):
            h = h + t @ Ws1_r[j * _F:(j + 1) * _F, :]
        h = jnp.maximum(h, 0.0)
        gv = h @ Ws2_r[...] + bs2_r[...]
        ged_r[g] = jnp.broadcast_to(jax.nn.sigmoid(gv), (1, 128))


def kernel(x1, cent1, rw1, src1, dst1, batch1,
           x2, cent2, rw2, src2, dst2, batch2,
           degree_emb, W_init, b_init, Wc1, bc1, Wc2, bc2, Wc3, bc3,
           A_aff, Ws1, bs1, Ws2, bs2):
    del batch1, batch2
    f32 = jnp.float32
    Wx = W_init[:_XDIM]
    We = W_init[_XDIM:_XDIM + _MAXDEG]
    Wr = W_init[_XDIM + _MAXDEG:]
    b0 = b_init.reshape(1, _F)
    bc1r = bc1.reshape(1, _F)
    bc2r = bc2.reshape(1, _F)
    bc3r = bc3.reshape(1, _F)
    bs1r = bs1.reshape(1, 64)
    bs2r = bs2.reshape(1, 1)
    s1 = src1.reshape(_B, 1, _EPG)
    d1 = dst1.reshape(_B, 1, _EPG)
    s2 = src2.reshape(_B, 1, _EPG)
    d2 = dst2.reshape(_B, 1, _EPG)

    node = lambda dim: pl.BlockSpec((_G * _NPG, dim), lambda i: (i, 0))
    edge = pl.BlockSpec((_G, 1, _EPG), lambda i: (i, 0, 0))
    full = lambda a: pl.BlockSpec(a.shape, lambda i: (0,) * a.ndim)

    in_specs = [
        node(_XDIM), node(1), node(_RWDIM), edge, edge,
        node(_XDIM), node(1), node(_RWDIM), edge, edge,
        full(degree_emb), full(Wx), full(We), full(Wr), full(b0),
        full(Wc1), full(bc1r), full(Wc2), full(bc2r), full(Wc3), full(bc3r),
        full(A_aff), full(Ws1), full(bs1r), full(Ws2), full(bs2r),
    ]
    out_specs = [
        pl.BlockSpec((_G, 1, 128), lambda i: (i, 0, 0)),
        pl.BlockSpec((_G, _NPG, _NPG), lambda i: (i, 0, 0)),
        pl.BlockSpec((_G, _NPG, _NPG), lambda i: (i, 0, 0)),
    ]
    out_shape = [
        jax.ShapeDtypeStruct((_B, 1, 128), f32),
        jax.ShapeDtypeStruct((_B, _NPG, _NPG), f32),
        jax.ShapeDtypeStruct((_B, _NPG, _NPG), f32),
    ]

    ged3, sim1, sim2 = pl.pallas_call(
        _group_body,
        grid=(_B // _G,),
        in_specs=in_specs,
        out_specs=out_specs,
        out_shape=out_shape,
        compiler_params=pltpu.CompilerParams(
            dimension_semantics=("arbitrary",)),
    )(x1, cent1, rw1, s1, d1, x2, cent2, rw2, s2, d2,
      degree_emb, Wx, We, Wr, b0, Wc1, bc1r, Wc2, bc2r, Wc3, bc3r,
      A_aff, Ws1, bs1r, Ws2, bs2r)
    return ged3[:, 0, 0], sim1, sim2


# speedup vs baseline: 28.2951x; 1.3720x over previous
import math

import jax
import jax.numpy as jnp
from jax.experimental import pallas as pl
from jax.experimental.pallas import tpu as pltpu

_B = 512
_G = 4
_NPG = 64
_EPG = 1024
_XDIM = 29
_MAXDEG = 83
_RWDIM = 16
_F = 128

_MU = 1.0 / 4096.0
_NU0 = 4088.0 / 4096.0
_NU1 = 8.0 / 4096.0


def _group_body(x1_r, c1_r, rw1_r, s1_r, d1_r,
                x2_r, c2_r, rw2_r, s2_r, d2_r,
                emb_r, Wx_r, We_r, Wr_r, b0_r,
                Wc1_r, bc1_r, Wc2_r, bc2_r, Wc3_r, bc3_r,
                Aaff_r, Ws1_r, bs1_r, Ws2_r, bs2_r,
                ged_r, sim1_r, sim2_r):
    f32 = jnp.float32

    def embed(x, cent, rw):
        oh = (cent == jax.lax.broadcasted_iota(jnp.int32, (_NPG, _MAXDEG), 1)
              ).astype(f32)
        e = (oh @ emb_r[...]) @ We_r[...]
        f = x @ Wx_r[...] + e + rw @ Wr_r[...] + b0_r[...]
        return jnp.maximum(f, 0.0)

    def adjacency(s, d):
        rows = jax.lax.broadcasted_iota(jnp.int32, (_NPG, _EPG), 0)
        ohS = (jnp.broadcast_to(s & 63, (_NPG, _EPG)) == rows).astype(f32)
        ohD = (jnp.broadcast_to(d & 63, (_NPG, _EPG)) == rows).astype(f32)
        counts = jax.lax.dot_general(ohD, ohS, (((1,), (1,)), ((), ())),
                                     preferred_element_type=f32,
                                     precision=jax.lax.Precision.DEFAULT)
        deg = jnp.sum(ohD, axis=1, keepdims=True) + 1.0
        dinv = jax.lax.rsqrt(deg)
        dd = 1.0 / deg
        return counts, dinv, dd

    def gcn(h, counts, dinv, dd, W_r, b_r):
        hw = h @ W_r[...]
        agg = jax.lax.dot_general(counts, hw * dinv, (((1,), (0,)), ((), ())),
                                  preferred_element_type=f32)
        return agg * dinv + hw * dd + b_r[...]

    def sim(xd, yd):
        t = xd @ Aaff_r[...]
        return jax.lax.dot_general(t, yd, (((1,), (1,)), ((), ())),
                                   preferred_element_type=f32)

    def soft_topk_multi(Ms):
        n = len(Ms)
        smins = [jnp.min(M) for M in Ms]
        smaxs = [jnp.max(M) for M in Ms]
        Rs = [jnp.exp(2.0 * ((M - lo) / (hi - lo + 1e-8)) - 1.0)
              for M, lo, hi in zip(Ms, smins, smaxs)]
        w0s = [1.0] * n
        w1s = [1.0] * n
        iDs = [None] * n
        for _ in range(6):
            for j in range(n):
                iDs[j] = 1.0 / (Rs[j] * w1s[j] + w0s[j])
            sum0 = [jnp.sum(iDs[j]) for j in range(n)]
            sum1 = [jnp.sum(Rs[j] * iDs[j]) for j in range(n)]
            for j in range(n):
                w0s[j] = _NU0 / (sum0[j] * _MU)
                w1s[j] = _NU1 / (sum1[j] * _MU)
        return [Rs[j] * iDs[j] * w1s[j] for j in range(n)]

    sims = []
    for g in range(_G):
        r0 = g * _NPG
        feat1 = embed(x1_r[r0:r0 + _NPG], c1_r[r0:r0 + _NPG],
                      rw1_r[r0:r0 + _NPG])
        feat2 = embed(x2_r[r0:r0 + _NPG], c2_r[r0:r0 + _NPG],
                      rw2_r[r0:r0 + _NPG])
        counts1, dinv1, dd1 = adjacency(s1_r[g], d1_r[g])
        counts2, dinv2, dd2 = adjacency(s2_r[g], d2_r[g])

        a11 = gcn(feat1, counts1, dinv1, dd1, Wc1_r, bc1_r)
        a12 = gcn(jnp.maximum(a11, 0.0), counts1, dinv1, dd1, Wc2_r, bc2_r)
        a13 = gcn(jnp.maximum(a12, 0.0), counts1, dinv1, dd1, Wc3_r, bc3_r)
        a21 = gcn(feat2, counts2, dinv2, dd2, Wc1_r, bc1_r)
        a22 = gcn(jnp.maximum(a21, 0.0), counts2, dinv2, dd2, Wc2_r, bc2_r)
        a23 = gcn(jnp.maximum(a22, 0.0), counts2, dinv2, dd2, Wc3_r, bc3_r)

        sims.append((sim(feat1, feat2), sim(a13, a23)))

        pooled = [jnp.sum(t, axis=0, keepdims=True)
                  for t in (feat1, a11, a12, a13)]
        pooled += [jnp.max(t, axis=0, keepdims=True)
                   for t in (feat2, a21, a22, a23)]
        h = bs1_r[...]
        for j, t in enumerate(pooled):
            h = h + t @ Ws1_r[j * _F:(j + 1) * _F, :]
        h = jnp.maximum(h, 0.0)
        gv = h @ Ws2_r[...] + bs2_r[...]
        ged_r[g] = jnp.broadcast_to(jax.nn.sigmoid(gv), (1, 128))

    probs = soft_topk_multi([m for pair in sims for m in pair])
    for g in range(_G):
        sim1_r[g] = probs[2 * g]
        sim2_r[g] = probs[2 * g + 1]


def kernel(x1, cent1, rw1, src1, dst1, batch1,
           x2, cent2, rw2, src2, dst2, batch2,
           degree_emb, W_init, b_init, Wc1, bc1, Wc2, bc2, Wc3, bc3,
           A_aff, Ws1, bs1, Ws2, bs2):
    del batch1, batch2
    f32 = jnp.float32
    Wx = W_init[:_XDIM]
    We = W_init[_XDIM:_XDIM + _MAXDEG]
    Wr = W_init[_XDIM + _MAXDEG:]
    b0 = b_init.reshape(1, _F)
    bc1r = bc1.reshape(1, _F)
    bc2r = bc2.reshape(1, _F)
    bc3r = bc3.reshape(1, _F)
    bs1r = bs1.reshape(1, 64)
    bs2r = bs2.reshape(1, 1)
    s1 = src1.reshape(_B, 1, _EPG)
    d1 = dst1.reshape(_B, 1, _EPG)
    s2 = src2.reshape(_B, 1, _EPG)
    d2 = dst2.reshape(_B, 1, _EPG)

    node = lambda dim: pl.BlockSpec((_G * _NPG, dim), lambda i: (i, 0))
    edge = pl.BlockSpec((_G, 1, _EPG), lambda i: (i, 0, 0))
    full = lambda a: pl.BlockSpec(a.shape, lambda i: (0,) * a.ndim)

    in_specs = [
        node(_XDIM), node(1), node(_RWDIM), edge, edge,
        node(_XDIM), node(1), node(_RWDIM), edge, edge,
        full(degree_emb), full(Wx), full(We), full(Wr), full(b0),
        full(Wc1), full(bc1r), full(Wc2), full(bc2r), full(Wc3), full(bc3r),
        full(A_aff), full(Ws1), full(bs1r), full(Ws2), full(bs2r),
    ]
    out_specs = [
        pl.BlockSpec((_G, 1, 128), lambda i: (i, 0, 0)),
        pl.BlockSpec((_G, _NPG, _NPG), lambda i: (i, 0, 0)),
        pl.BlockSpec((_G, _NPG, _NPG), lambda i: (i, 0, 0)),
    ]
    out_shape = [
        jax.ShapeDtypeStruct((_B, 1, 128), f32),
        jax.ShapeDtypeStruct((_B, _NPG, _NPG), f32),
        jax.ShapeDtypeStruct((_B, _NPG, _NPG), f32),
    ]

    ged3, sim1, sim2 = pl.pallas_call(
        _group_body,
        grid=(_B // _G,),
        in_specs=in_specs,
        out_specs=out_specs,
        out_shape=out_shape,
        compiler_params=pltpu.CompilerParams(
            dimension_semantics=("arbitrary",)),
    )(x1, cent1, rw1, s1, d1, x2, cent2, rw2, s2, d2,
      degree_emb, Wx, We, Wr, b0, Wc1, bc1r, Wc2, bc2r, Wc3, bc3r,
      A_aff, Ws1, bs1r, Ws2, bs2r)
    return ged3[:, 0, 0], sim1, sim2


# speedup vs baseline: 90.8988x; 3.2125x over previous
import math

import jax
import jax.numpy as jnp
from jax.experimental import pallas as pl
from jax.experimental.pallas import tpu as pltpu

_B = 512
_G = 4
_NPG = 64
_EPG = 1024
_XDIM = 29
_MAXDEG = 83
_RWDIM = 16
_F = 128

_MU = 1.0 / 4096.0
_NU0 = 4088.0 / 4096.0
_NU1 = 8.0 / 4096.0


def _group_body(x1_r, c1_r, rw1_r, s1_r, d1_r,
                x2_r, c2_r, rw2_r, s2_r, d2_r,
                emb_r, Wx_r, We_r, Wr_r, b0_r,
                Wc1_r, bc1_r, Wc2_r, bc2_r, Wc3_r, bc3_r,
                Aaff_r, Ws1_r, bs1_r, Ws2_r, bs2_r,
                ged_r, sim1_r, sim2_r):
    f32 = jnp.float32

    def embed(x, cent, rw):
        oh = (cent == jax.lax.broadcasted_iota(jnp.int32, (_NPG, _MAXDEG), 1)
              ).astype(f32)
        e = (oh @ emb_r[...]) @ We_r[...]
        f = x @ Wx_r[...] + e + rw @ Wr_r[...] + b0_r[...]
        return jnp.maximum(f, 0.0)

    def adjacency(s, d):
        rows = jax.lax.broadcasted_iota(jnp.int32, (_NPG, _EPG), 0)
        ohS = (jnp.broadcast_to(s & 63, (_NPG, _EPG)) == rows).astype(f32)
        ohD = (jnp.broadcast_to(d & 63, (_NPG, _EPG)) == rows).astype(f32)
        counts = jax.lax.dot_general(ohD, ohS, (((1,), (1,)), ((), ())),
                                     preferred_element_type=f32,
                                     precision=jax.lax.Precision.DEFAULT)
        deg = jnp.sum(ohD, axis=1, keepdims=True) + 1.0
        dinv = jax.lax.rsqrt(deg)
        dd = 1.0 / deg
        return counts, dinv, dd

    def gcn(h, counts, dinv, dd, W_r, b_r):
        hw = h @ W_r[...]
        agg = jax.lax.dot_general(counts, hw * dinv, (((1,), (0,)), ((), ())),
                                  preferred_element_type=f32)
        return agg * dinv + hw * dd + b_r[...]

    def sim(xd, yd):
        t = jnp.dot(xd, Aaff_r[...], precision=jax.lax.Precision.HIGHEST)
        return jax.lax.dot_general(t, yd, (((1,), (1,)), ((), ())),
                                   preferred_element_type=f32,
                                   precision=jax.lax.Precision.HIGHEST)

    def soft_topk_multi(Ms):
        n = len(Ms)
        smins = [jnp.min(M) for M in Ms]
        smaxs = [jnp.max(M) for M in Ms]
        Rs = [jnp.exp(2.0 * ((M - lo) / (hi - lo + 1e-8)) - 1.0)
              for M, lo, hi in zip(Ms, smins, smaxs)]
        w0s = [1.0] * n
        w1s = [1.0] * n
        iDs = [None] * n
        for _ in range(6):
            for j in range(n):
                iDs[j] = 1.0 / (Rs[j] * w1s[j] + w0s[j])
            sum0 = [jnp.sum(iDs[j]) for j in range(n)]
            sum1 = [jnp.sum(Rs[j] * iDs[j]) for j in range(n)]
            for j in range(n):
                w0s[j] = _NU0 / (sum0[j] * _MU)
                w1s[j] = _NU1 / (sum1[j] * _MU)
        return [Rs[j] * iDs[j] * w1s[j] for j in range(n)]

    ng = 2 * _G
    nr = ng * _NPG

    xa = jnp.concatenate([x1_r[...], x2_r[...]], axis=0)
    ca = jnp.concatenate([c1_r[...], c2_r[...]], axis=0)
    rwa = jnp.concatenate([rw1_r[...], rw2_r[...]], axis=0)
    oh = (ca == jax.lax.broadcasted_iota(jnp.int32, (nr, _MAXDEG), 1)
          ).astype(f32)
    feat = jnp.maximum(xa @ Wx_r[...] + (oh @ emb_r[...]) @ We_r[...]
                       + rwa @ Wr_r[...] + b0_r[...], 0.0)

    adjs = [adjacency(s1_r[g], d1_r[g]) for g in range(_G)]
    adjs += [adjacency(s2_r[g], d2_r[g]) for g in range(_G)]
    dinv_all = jnp.concatenate([a[1] for a in adjs], axis=0)
    dd_all = jnp.concatenate([a[2] for a in adjs], axis=0)

    def gcn_all(h, W_r, b_r):
        hw = h @ W_r[...]
        hs = hw * dinv_all
        aggs = [jax.lax.dot_general(
            adjs[k][0], hs[k * _NPG:(k + 1) * _NPG],
            (((1,), (0,)), ((), ())), preferred_element_type=f32)
            for k in range(ng)]
        return (jnp.concatenate(aggs, axis=0) * dinv_all
                + hw * dd_all + b_r[...])

    a1 = gcn_all(feat, Wc1_r, bc1_r)
    a2 = gcn_all(jnp.maximum(a1, 0.0), Wc2_r, bc2_r)
    a3 = gcn_all(jnp.maximum(a2, 0.0), Wc3_r, bc3_r)

    half = _G * _NPG
    hp = jax.lax.Precision.HIGHEST
    t1 = jnp.dot(feat[:half], Aaff_r[...], precision=hp)
    t2 = jnp.dot(a3[:half], Aaff_r[...], precision=hp)
    sims = []
    for g in range(_G):
        r0 = g * _NPG
        sims.append(jax.lax.dot_general(
            t1[r0:r0 + _NPG], feat[half + r0:half + r0 + _NPG],
            (((1,), (1,)), ((), ())), preferred_element_type=f32,
            precision=hp))
        sims.append(jax.lax.dot_general(
            t2[r0:r0 + _NPG], a3[half + r0:half + r0 + _NPG],
            (((1,), (1,)), ((), ())), preferred_element_type=f32,
            precision=hp))

    probs = soft_topk_multi(sims)
    for g in range(_G):
        sim1_r[g] = probs[2 * g]
        sim2_r[g] = probs[2 * g + 1]

    rows = []
    for g in range(_G):
        r0 = g * _NPG
        ps = [jnp.sum(t[r0:r0 + _NPG], axis=0, keepdims=True)
              for t in (feat, a1, a2, a3)]
        qs = [jnp.max(t[half + r0:half + r0 + _NPG], axis=0, keepdims=True)
              for t in (feat, a1, a2, a3)]
        rows.append(jnp.concatenate(ps + qs, axis=1))
    scores = jnp.concatenate(rows, axis=0)
    hsc = jnp.maximum(scores @ Ws1_r[...] + bs1_r[...], 0.0)
    gv = jax.nn.sigmoid(hsc @ Ws2_r[...] + bs2_r[...])
    ged_r[...] = jnp.broadcast_to(gv[:, None, :], (_G, 1, 128))


def kernel(x1, cent1, rw1, src1, dst1, batch1,
           x2, cent2, rw2, src2, dst2, batch2,
           degree_emb, W_init, b_init, Wc1, bc1, Wc2, bc2, Wc3, bc3,
           A_aff, Ws1, bs1, Ws2, bs2):
    del batch1, batch2
    f32 = jnp.float32
    Wx = W_init[:_XDIM]
    We = W_init[_XDIM:_XDIM + _MAXDEG]
    Wr = W_init[_XDIM + _MAXDEG:]
    b0 = b_init.reshape(1, _F)
    bc1r = bc1.reshape(1, _F)
    bc2r = bc2.reshape(1, _F)
    bc3r = bc3.reshape(1, _F)
    bs1r = bs1.reshape(1, 64)
    bs2r = bs2.reshape(1, 1)
    s1 = src1.reshape(_B, 1, _EPG)
    d1 = dst1.reshape(_B, 1, _EPG)
    s2 = src2.reshape(_B, 1, _EPG)
    d2 = dst2.reshape(_B, 1, _EPG)

    node = lambda dim: pl.BlockSpec((_G * _NPG, dim), lambda i: (i, 0))
    edge = pl.BlockSpec((_G, 1, _EPG), lambda i: (i, 0, 0))
    full = lambda a: pl.BlockSpec(a.shape, lambda i: (0,) * a.ndim)

    in_specs = [
        node(_XDIM), node(1), node(_RWDIM), edge, edge,
        node(_XDIM), node(1), node(_RWDIM), edge, edge,
        full(degree_emb), full(Wx), full(We), full(Wr), full(b0),
        full(Wc1), full(bc1r), full(Wc2), full(bc2r), full(Wc3), full(bc3r),
        full(A_aff), full(Ws1), full(bs1r), full(Ws2), full(bs2r),
    ]
    out_specs = [
        pl.BlockSpec((_G, 1, 128), lambda i: (i, 0, 0)),
        pl.BlockSpec((_G, _NPG, _NPG), lambda i: (i, 0, 0)),
        pl.BlockSpec((_G, _NPG, _NPG), lambda i: (i, 0, 0)),
    ]
    out_shape = [
        jax.ShapeDtypeStruct((_B, 1, 128), f32),
        jax.ShapeDtypeStruct((_B, _NPG, _NPG), f32),
        jax.ShapeDtypeStruct((_B, _NPG, _NPG), f32),
    ]

    ged3, sim1, sim2 = pl.pallas_call(
        _group_body,
        grid=(_B // _G,),
        in_specs=in_specs,
        out_specs=out_specs,
        out_shape=out_shape,
        compiler_params=pltpu.CompilerParams(
            dimension_semantics=("arbitrary",)),
    )(x1, cent1, rw1, s1, d1, x2, cent2, rw2, s2, d2,
      degree_emb, Wx, We, Wr, b0, Wc1, bc1r, Wc2, bc2r, Wc3, bc3r,
      A_aff, Ws1, bs1r, Ws2, bs2r)
    return ged3[:, 0, 0], sim1, sim2


# speedup vs baseline: 93.9815x; 1.0339x over previous
import functools
import math

import jax
import jax.numpy as jnp
from jax import lax
from jax.experimental import pallas as pl
from jax.experimental.pallas import tpu as pltpu
from jax.experimental.pallas import tpu_sc as plsc

_B = 512
_G = 16
_NPG = 64
_EPG = 1024
_XDIM = 29
_MAXDEG = 83
_RWDIM = 16
_F = 128

_MU = 1.0 / 4096.0
_NU0 = 4088.0 / 4096.0
_NU1 = 8.0 / 4096.0


_EG = 2 * _B
_NW = 32
_GPW = _EG // _NW


@functools.partial(
    pl.kernel,
    out_type=jax.ShapeDtypeStruct((_EG, _NPG * _NPG), jnp.float32),
    mesh=plsc.VectorSubcoreMesh(core_axis_name="c", subcore_axis_name="s",
                                num_cores=2, num_subcores=16),
    scratch_types=[
        pltpu.VMEM((_EPG,), jnp.int32),
        pltpu.VMEM((_EPG,), jnp.int32),
        pltpu.VMEM((_NPG * _NPG,), jnp.float32),
    ],
    compiler_params=pltpu.CompilerParams(needs_layout_passes=False),
)
def _sc_counts(src_hbm, dst_hbm, out_hbm, s_v, d_v, buf):
    wid = lax.axis_index("s") * 2 + lax.axis_index("c")
    zeros16 = jnp.zeros((16,), jnp.float32)
    ones16 = jnp.ones((16,), jnp.float32)

    def per_graph(gi, carry):
        g = wid * _GPW + gi
        base = g * _EPG
        pltpu.sync_copy(src_hbm.at[pl.ds(base, _EPG)], s_v)
        pltpu.sync_copy(dst_hbm.at[pl.ds(base, _EPG)], d_v)
        for i in range(_NPG * _NPG // 16):
            buf[pl.ds(i * 16, 16)] = zeros16
        for j in range(_EPG // 16):
            s = s_v[pl.ds(j * 16, 16)]
            d = d_v[pl.ds(j * 16, 16)]
            key = ((d & 63) << 6) | (s & 63)
            plsc.addupdate_scatter(buf, [key], ones16)
        pltpu.sync_copy(buf, out_hbm.at[g])
        return carry

    lax.fori_loop(0, _GPW, per_graph, 0)


def _group_body(x1_r, c1_r, rw1_r, ct1_r,
                x2_r, c2_r, rw2_r, ct2_r,
                emb_r, Wx_r, We_r, Wr_r, b0_r,
                Wc1_r, bc1_r, Wc2_r, bc2_r, Wc3_r, bc3_r,
                Aaff_r, Ws1_r, bs1_r, Ws2_r, bs2_r,
                ged_r, sim1_r, sim2_r):
    f32 = jnp.float32

    def adjacency(counts):
        deg = jnp.sum(counts, axis=1, keepdims=True) + 1.0
        dinv = jax.lax.rsqrt(deg)
        dd = 1.0 / deg
        return counts, dinv, dd

    def soft_topk_multi(Ms):
        n = len(Ms)
        smins = [jnp.min(M) for M in Ms]
        smaxs = [jnp.max(M) for M in Ms]
        Rs = [jnp.exp(2.0 * ((M - lo) / (hi - lo + 1e-8)) - 1.0)
              for M, lo, hi in zip(Ms, smins, smaxs)]
        w0s = [1.0] * n
        w1s = [1.0] * n
        iDs = [None] * n
        for _ in range(6):
            for j in range(n):
                iDs[j] = 1.0 / (Rs[j] * w1s[j] + w0s[j])
            sum0 = [jnp.sum(iDs[j]) for j in range(n)]
            sum1 = [jnp.sum(Rs[j] * iDs[j]) for j in range(n)]
            for j in range(n):
                w0s[j] = _NU0 / (sum0[j] * _MU)
                w1s[j] = _NU1 / (sum1[j] * _MU)
        return [Rs[j] * iDs[j] * w1s[j] for j in range(n)]

    ng = 2 * _G
    nr = ng * _NPG

    xa = jnp.concatenate([x1_r[...], x2_r[...]], axis=0)
    ca = jnp.concatenate([c1_r[...], c2_r[...]], axis=0)
    rwa = jnp.concatenate([rw1_r[...], rw2_r[...]], axis=0)
    oh = (ca == jax.lax.broadcasted_iota(jnp.int32, (nr, _MAXDEG), 1)
          ).astype(f32)
    feat = jnp.maximum(xa @ Wx_r[...] + (oh @ emb_r[...]) @ We_r[...]
                       + rwa @ Wr_r[...] + b0_r[...], 0.0)

    adjs = [adjacency(ct1_r[g]) for g in range(_G)]
    adjs += [adjacency(ct2_r[g]) for g in range(_G)]
    dinv_all = jnp.concatenate([a[1] for a in adjs], axis=0)
    dd_all = jnp.concatenate([a[2] for a in adjs], axis=0)

    def gcn_all(h, W_r, b_r):
        hw = h @ W_r[...]
        hs = hw * dinv_all
        aggs = [jax.lax.dot_general(
            adjs[k][0], hs[k * _NPG:(k + 1) * _NPG],
            (((1,), (0,)), ((), ())), preferred_element_type=f32)
            for k in range(ng)]
        return (jnp.concatenate(aggs, axis=0) * dinv_all
                + hw * dd_all + b_r[...])

    a1 = gcn_all(feat, Wc1_r, bc1_r)
    a2 = gcn_all(jnp.maximum(a1, 0.0), Wc2_r, bc2_r)
    a3 = gcn_all(jnp.maximum(a2, 0.0), Wc3_r, bc3_r)

    half = _G * _NPG
    t1 = feat[:half] @ Aaff_r[...]
    t2 = a3[:half] @ Aaff_r[...]
    sims = []
    for g in range(_G):
        r0 = g * _NPG
        sims.append(jax.lax.dot_general(
            t1[r0:r0 + _NPG], feat[half + r0:half + r0 + _NPG],
            (((1,), (1,)), ((), ())), preferred_element_type=f32))
        sims.append(jax.lax.dot_general(
            t2[r0:r0 + _NPG], a3[half + r0:half + r0 + _NPG],
            (((1,), (1,)), ((), ())), preferred_element_type=f32))

    probs = soft_topk_multi(sims)
    for g in range(_G):
        sim1_r[g] = probs[2 * g]
        sim2_r[g] = probs[2 * g + 1]

    rows = []
    for g in range(_G):
        r0 = g * _NPG
        ps = [jnp.sum(t[r0:r0 + _NPG], axis=0, keepdims=True)
              for t in (feat, a1, a2, a3)]
        qs = [jnp.max(t[half + r0:half + r0 + _NPG], axis=0, keepdims=True)
              for t in (feat, a1, a2, a3)]
        rows.append(jnp.concatenate(ps + qs, axis=1))
    scores = jnp.concatenate(rows, axis=0)
    hsc = jnp.maximum(scores @ Ws1_r[...] + bs1_r[...], 0.0)
    gv = jax.nn.sigmoid(hsc @ Ws2_r[...] + bs2_r[...])
    ged_r[...] = jnp.broadcast_to(gv[:, None, :], (_G, 1, 128))


def kernel(x1, cent1, rw1, src1, dst1, batch1,
           x2, cent2, rw2, src2, dst2, batch2,
           degree_emb, W_init, b_init, Wc1, bc1, Wc2, bc2, Wc3, bc3,
           A_aff, Ws1, bs1, Ws2, bs2):
    del batch1, batch2
    f32 = jnp.float32
    Wx = W_init[:_XDIM]
    We = W_init[_XDIM:_XDIM + _MAXDEG]
    Wr = W_init[_XDIM + _MAXDEG:]
    b0 = b_init.reshape(1, _F)
    bc1r = bc1.reshape(1, _F)
    bc2r = bc2.reshape(1, _F)
    bc3r = bc3.reshape(1, _F)
    bs1r = bs1.reshape(1, 64)
    bs2r = bs2.reshape(1, 1)
    src_all = jnp.concatenate([src1, src2])
    dst_all = jnp.concatenate([dst1, dst2])
    counts = _sc_counts(src_all, dst_all).reshape(_EG, _NPG, _NPG)

    node = lambda dim: pl.BlockSpec((_G * _NPG, dim), lambda i: (i, 0))
    cts1 = pl.BlockSpec((_G, _NPG, _NPG), lambda i: (i, 0, 0))
    cts2 = pl.BlockSpec((_G, _NPG, _NPG), lambda i: (i + _B // _G, 0, 0))
    full = lambda a: pl.BlockSpec(a.shape, lambda i: (0,) * a.ndim)

    in_specs = [
        node(_XDIM), node(1), node(_RWDIM), cts1,
        node(_XDIM), node(1), node(_RWDIM), cts2,
        full(degree_emb), full(Wx), full(We), full(Wr), full(b0),
        full(Wc1), full(bc1r), full(Wc2), full(bc2r), full(Wc3), full(bc3r),
        full(A_aff), full(Ws1), full(bs1r), full(Ws2), full(bs2r),
    ]
    out_specs = [
        pl.BlockSpec((_G, 1, 128), lambda i: (i, 0, 0)),
        pl.BlockSpec((_G, _NPG, _NPG), lambda i: (i, 0, 0)),
        pl.BlockSpec((_G, _NPG, _NPG), lambda i: (i, 0, 0)),
    ]
    out_shape = [
        jax.ShapeDtypeStruct((_B, 1, 128), f32),
        jax.ShapeDtypeStruct((_B, _NPG, _NPG), f32),
        jax.ShapeDtypeStruct((_B, _NPG, _NPG), f32),
    ]

    ged3, sim1, sim2 = pl.pallas_call(
        _group_body,
        grid=(_B // _G,),
        in_specs=in_specs,
        out_specs=out_specs,
        out_shape=out_shape,
        compiler_params=pltpu.CompilerParams(
            dimension_semantics=("arbitrary",)),
    )(x1, cent1, rw1, counts, x2, cent2, rw2, counts,
      degree_emb, Wx, We, Wr, b0, Wc1, bc1r, Wc2, bc2r, Wc3, bc3r,
      A_aff, Ws1, bs1r, Ws2, bs2r)
    return ged3[:, 0, 0], sim1, sim2
